# baseline (speedup 1.0000x reference)
import jax
import jax.numpy as jnp
from jax.experimental import pallas as pl
from jax.experimental.pallas import tpu as pltpu

_BB = 128


def _add_kernel(x_ref, e_ref, o_ref):
    o_ref[...] = x_ref[...] + e_ref[...][None, :, :]


def kernel(x, encoding):
    B, L, D = x.shape
    grid = (B // _BB,)
    return pl.pallas_call(
        _add_kernel,
        grid=grid,
        in_specs=[
            pl.BlockSpec((_BB, L, D), lambda i: (i, 0, 0)),
            pl.BlockSpec((L, D), lambda i: (0, 0)),
        ],
        out_specs=pl.BlockSpec((_BB, L, D), lambda i: (i, 0, 0)),
        out_shape=jax.ShapeDtypeStruct((B, L, D), x.dtype),
        compiler_params=pltpu.CompilerParams(
            dimension_semantics=("parallel",),
        ),
    )(x, encoding)


# speedup vs baseline: 4.6216x; 4.6216x over previous
import jax
import jax.numpy as jnp
from jax.experimental import pallas as pl
from jax.experimental.pallas import tpu as pltpu

_RR = 400


def _add_kernel(x_ref, pe_ref, o_ref):
    o_ref[...] = x_ref[...] + pe_ref[...]


def kernel(x, encoding):
    B, L, D = x.shape
    LD = L * D
    x2 = x.transpose(1, 2, 0).reshape(LD, B)
    pe = encoding.reshape(encoding.shape[0] * D, 1)
    grid = (LD // _RR,)
    out = pl.pallas_call(
        _add_kernel,
        grid=grid,
        in_specs=[
            pl.BlockSpec((_RR, B), lambda i: (i, 0)),
            pl.BlockSpec((_RR, 1), lambda i: (i, 0)),
        ],
        out_specs=pl.BlockSpec((_RR, B), lambda i: (i, 0)),
        out_shape=jax.ShapeDtypeStruct((LD, B), x.dtype),
        compiler_params=pltpu.CompilerParams(
            dimension_semantics=("parallel",),
        ),
    )(x2, pe)
    return out.reshape(L, D, B).transpose(2, 0, 1)


# speedup vs baseline: 5.8475x; 1.2653x over previous
import jax
import jax.numpy as jnp
from jax.experimental import pallas as pl
from jax.experimental.pallas import tpu as pltpu

_RR = 400


def _add_kernel(x_ref, pe_ref, o_ref):
    o_ref[...] = x_ref[...] + pe_ref[...]


def kernel(x, encoding):
    B, L, D = x.shape
    LD = L * D
    x2 = x.transpose(1, 2, 0).reshape(LD, B)
    pe = encoding[:L].reshape(LD, 1)
    grid = (LD // _RR,)
    out = pl.pallas_call(
        _add_kernel,
        grid=grid,
        in_specs=[
            pl.BlockSpec((_RR, B), lambda i: (i, 0)),
            pl.BlockSpec((_RR, 1), lambda i: (i, 0)),
        ],
        out_specs=pl.BlockSpec((_RR, B), lambda i: (i, 0)),
        out_shape=jax.ShapeDtypeStruct((LD, B), x.dtype),
        compiler_params=pltpu.CompilerParams(
            dimension_semantics=("parallel",),
        ),
    )(x2, pe)
    return out.reshape(L, D, B).transpose(2, 0, 1)


# speedup vs baseline: 5.8590x; 1.0020x over previous
import jax
import jax.numpy as jnp
from jax.experimental import pallas as pl
from jax.experimental.pallas import tpu as pltpu

_RR = 800


def _add_kernel(x_ref, pe_ref, o_ref):
    o_ref[...] = x_ref[...] + pe_ref[...]


def kernel(x, encoding):
    B, L, D = x.shape
    LD = L * D
    x2 = x.transpose(1, 2, 0).reshape(LD, B)
    pe = encoding[:L].reshape(LD, 1)
    grid = (LD // _RR,)
    out = pl.pallas_call(
        _add_kernel,
        grid=grid,
        in_specs=[
            pl.BlockSpec((_RR, B), lambda i: (i, 0)),
            pl.BlockSpec((_RR, 1), lambda i: (i, 0)),
        ],
        out_specs=pl.BlockSpec((_RR, B), lambda i: (i, 0)),
        out_shape=jax.ShapeDtypeStruct((LD, B), x.dtype),
        compiler_params=pltpu.CompilerParams(
            dimension_semantics=("parallel",),
            vmem_limit_bytes=110 * 1024 * 1024,
        ),
    )(x2, pe)
    return out.reshape(L, D, B).transpose(2, 0, 1)
